# baseline (speedup 1.0000x reference)
import jax
import jax.numpy as jnp
from jax import lax
from jax.experimental import pallas as pl
from jax.experimental.pallas import tpu as pltpu
from jax.experimental.pallas import tpu_sc as plsc

VOCAB = 100004
DIM = 200
BATCH = 4096
SEQ = 50

NC = 2
NS = 16
NW = NC * NS

BPW = BATCH // NW
DBLK = DIM // 8
L = 16

TBLK = 512
DIM_A = 128
DIM_B = DIM - DIM_A


def _transpose_body(in_ref, out_a_ref, out_b_ref):
    blk = in_ref[...]
    out_a_ref[...] = blk[:DIM_A, :].T
    out_b_ref[:, :DIM_B] = blk[DIM_A:, :].T


@jax.jit
def _tc_transpose(t_t):
    return pl.pallas_call(
        _transpose_body,
        grid=(pl.cdiv(VOCAB, TBLK),),
        in_specs=[pl.BlockSpec((DIM, TBLK), lambda i: (0, i))],
        out_specs=[
            pl.BlockSpec((TBLK, DIM_A), lambda i: (i, 0)),
            pl.BlockSpec((TBLK, DIM_A), lambda i: (i, 0)),
        ],
        out_shape=[
            jax.ShapeDtypeStruct((VOCAB, DIM_A), jnp.float32),
            jax.ShapeDtypeStruct((VOCAB, DIM_A), jnp.float32),
        ],
    )(t_t)


def _body(ta_hbm, tb_hbm, idxt_hbm, out_hbm, idx_v,
          buf_a0, buf_a1, buf_b0, buf_b1, out_b0, out_b1,
          gsem0, gsem1, wsem0, wsem1):
    bufs_a = (buf_a0, buf_a1)
    bufs_b = (buf_b0, buf_b1)
    outbs = (out_b0, out_b1)
    gsems = (gsem0, gsem1)
    wsems = (wsem0, wsem1)

    c = lax.axis_index("c")
    s = lax.axis_index("s")
    wid = s * NC + c

    pltpu.sync_copy(idxt_hbm.at[:, pl.ds(wid * BPW, BPW)], idx_v)

    iota = lax.broadcasted_iota(jnp.int32, (L,), 0)
    row_idx = [iota + L * g for g in range(BPW // L)]

    def start_gathers(b, li):
        pltpu.make_async_copy(
            ta_hbm.at[idx_v.at[li]], bufs_a[b], gsems[b]).start()
        pltpu.make_async_copy(
            tb_hbm.at[idx_v.at[li]], bufs_b[b], gsems[b]).start()

    def wait_gathers(b, li):
        pltpu.make_async_copy(
            ta_hbm.at[idx_v.at[li]], bufs_a[b], gsems[b]).wait()
        pltpu.make_async_copy(
            tb_hbm.at[idx_v.at[li]], bufs_b[b], gsems[b]).wait()

    def out_copy(b, li):
        return pltpu.make_async_copy(
            outbs[b], out_hbm.at[li, :, wid], wsems[b])

    for b in range(2):
        start_gathers(b, b)

    @pl.loop(0, SEQ // 2)
    def _outer(o):
        for b in range(2):
            li = o * 2 + b
            wait_gathers(b, li)

            @pl.when(o >= 1)
            def _():
                out_copy(b, li - 2).wait()

            @pl.loop(0, DIM_A // 8)
            def _da(d):
                for dd in range(8):
                    col = jnp.full((L,), d * 8 + dd, jnp.int32)
                    for g in range(BPW // L):
                        v = plsc.load_gather(bufs_a[b], [row_idx[g], col])
                        outbs[b][d, dd, pl.ds(g * L, L)] = v

            @pl.loop(DIM_A // 8, DBLK)
            def _db(d):
                for dd in range(8):
                    col = jnp.full((L,), d * 8 + dd - DIM_A, jnp.int32)
                    for g in range(BPW // L):
                        v = plsc.load_gather(bufs_b[b], [row_idx[g], col])
                        outbs[b][d, dd, pl.ds(g * L, L)] = v

            out_copy(b, li).start()

            @pl.when(o < SEQ // 2 - 1)
            def _():
                start_gathers(b, li + 2)

    for b in range(2):
        out_copy(b, SEQ - 2 + b).wait()


@jax.jit
def _embed(idx_t, ta, tb):
    mesh = plsc.VectorSubcoreMesh(core_axis_name="c", subcore_axis_name="s")
    f = pl.kernel(
        _body,
        out_type=jax.ShapeDtypeStruct((SEQ, DBLK, NW, 8, DIM_A), jnp.float32),
        mesh=mesh,
        scratch_types=[
            pltpu.VMEM((SEQ, BPW), jnp.int32),
            *[pltpu.VMEM((BPW, DIM_A), jnp.float32) for _ in range(4)],
            *[pltpu.VMEM((DBLK, 8, DIM_A), jnp.float32) for _ in range(2)],
            *[pltpu.SemaphoreType.DMA for _ in range(4)],
        ],
        compiler_params=pltpu.CompilerParams(
            use_tc_tiling_on_sc=False, needs_layout_passes=False),
    )
    return f(ta, tb, idx_t)


def kernel(indices, table):
    ta, tb = _tc_transpose(table.T)
    idx_t = indices.T.astype(jnp.int32)
    out5 = _embed(idx_t, ta, tb)
    return out5.transpose(2, 4, 0, 1, 3).reshape(BATCH, SEQ, DIM)


# speedup vs baseline: 1.8815x; 1.8815x over previous
import jax
import jax.numpy as jnp
from jax import lax
from jax.experimental import pallas as pl
from jax.experimental.pallas import tpu as pltpu
from jax.experimental.pallas import tpu_sc as plsc

VOCAB = 100004
DIM = 200
BATCH = 4096
SEQ = 50

NC = 2
NS = 16
NW = NC * NS

BPW = BATCH // NW
B_TOTAL = BATCH * SEQ

TBLK = 512
DIM_A = 128
DIM_B = DIM - DIM_A
OBB = 512


def _transpose_body(in_ref, out_a_ref, out_b_ref):
    blk = in_ref[...]
    out_a_ref[...] = blk[:DIM_A, :].T
    out_b_ref[:, :DIM_B] = blk[DIM_A:, :].T


@jax.jit
def _tc_transpose(t_t):
    return pl.pallas_call(
        _transpose_body,
        grid=(pl.cdiv(VOCAB, TBLK),),
        in_specs=[pl.BlockSpec((DIM, TBLK), lambda i: (0, i))],
        out_specs=[
            pl.BlockSpec((TBLK, DIM_A), lambda i: (i, 0)),
            pl.BlockSpec((TBLK, DIM_A), lambda i: (i, 0)),
        ],
        out_shape=[
            jax.ShapeDtypeStruct((VOCAB, DIM_A), jnp.float32),
            jax.ShapeDtypeStruct((VOCAB, DIM_A), jnp.float32),
        ],
    )(t_t)


def _body(ta_hbm, tb_hbm, idxt_hbm, pa_hbm, pb_hbm, idx_v,
          buf_a0, buf_a1, buf_b0, buf_b1, gsem0, gsem1, wsem0, wsem1):
    bufs_a = (buf_a0, buf_a1)
    bufs_b = (buf_b0, buf_b1)
    gsems = (gsem0, gsem1)
    wsems = (wsem0, wsem1)

    c = lax.axis_index("c")
    s = lax.axis_index("s")
    wid = s * NC + c

    pltpu.sync_copy(idxt_hbm.at[:, pl.ds(wid * BPW, BPW)], idx_v)

    def start_gathers(b, li):
        pltpu.make_async_copy(
            ta_hbm.at[idx_v.at[li]], bufs_a[b], gsems[b]).start()
        pltpu.make_async_copy(
            tb_hbm.at[idx_v.at[li]], bufs_b[b], gsems[b]).start()

    def wait_gathers(b, li):
        pltpu.make_async_copy(
            ta_hbm.at[idx_v.at[li]], bufs_a[b], gsems[b]).wait()
        pltpu.make_async_copy(
            tb_hbm.at[idx_v.at[li]], bufs_b[b], gsems[b]).wait()

    for b in range(2):
        start_gathers(b, b)

    @pl.loop(0, SEQ // 2)
    def _outer(o):
        for b in range(2):
            li = o * 2 + b
            rows = pl.ds(li * BATCH + wid * BPW, BPW)
            wait_gathers(b, li)
            wa = pltpu.make_async_copy(bufs_a[b], pa_hbm.at[rows], wsems[b])
            wb = pltpu.make_async_copy(bufs_b[b], pb_hbm.at[rows], wsems[b])
            wa.start()
            wb.start()
            wa.wait()
            wb.wait()

            @pl.when(o < SEQ // 2 - 1)
            def _():
                start_gathers(b, li + 2)


@jax.jit
def _embed(idx_t, ta, tb):
    mesh = plsc.VectorSubcoreMesh(core_axis_name="c", subcore_axis_name="s")
    f = pl.kernel(
        _body,
        out_type=[
            jax.ShapeDtypeStruct((B_TOTAL, DIM_A), jnp.float32),
            jax.ShapeDtypeStruct((B_TOTAL, DIM_A), jnp.float32),
        ],
        mesh=mesh,
        scratch_types=[
            pltpu.VMEM((SEQ, BPW), jnp.int32),
            *[pltpu.VMEM((BPW, DIM_A), jnp.float32) for _ in range(4)],
            *[pltpu.SemaphoreType.DMA for _ in range(4)],
        ],
        compiler_params=pltpu.CompilerParams(use_tc_tiling_on_sc=False),
    )
    return f(ta, tb, idx_t)


def _out_transpose_body(a_ref, b_ref, o_ref):
    a2 = a_ref[0]
    b2 = b_ref[0]
    o_ref[0] = jnp.concatenate([a2, b2[:, :DIM_B]], axis=1).T


@jax.jit
def _tc_out(pa3, pb3):
    return pl.pallas_call(
        _out_transpose_body,
        grid=(SEQ, BATCH // OBB),
        in_specs=[
            pl.BlockSpec((1, OBB, DIM_A), lambda i, j: (i, j, 0)),
            pl.BlockSpec((1, OBB, DIM_A), lambda i, j: (i, j, 0)),
        ],
        out_specs=pl.BlockSpec((1, DIM, OBB), lambda i, j: (i, 0, j)),
        out_shape=jax.ShapeDtypeStruct((SEQ, DIM, BATCH), jnp.float32),
    )(pa3, pb3)


def kernel(indices, table):
    ta, tb = _tc_transpose(table.T)
    idx_t = indices.T.astype(jnp.int32)
    pa, pb = _embed(idx_t, ta, tb)
    out_t = _tc_out(pa.reshape(SEQ, BATCH, DIM_A), pb.reshape(SEQ, BATCH, DIM_A))
    return out_t.transpose(2, 0, 1)


# speedup vs baseline: 2.4305x; 1.2918x over previous
import jax
import jax.numpy as jnp
from jax import lax
from jax.experimental import pallas as pl
from jax.experimental.pallas import tpu as pltpu
from jax.experimental.pallas import tpu_sc as plsc

VOCAB = 100004
DIM = 200
BATCH = 4096
SEQ = 50

NC = 2
NS = 16
NW = NC * NS

BPW = BATCH // NW
B_TOTAL = BATCH * SEQ

TBLK = 1024
DIM_A = 128
DIM_B = DIM - DIM_A
OBB = 1024


def _transpose_body(in_ref, out_a_ref, out_b_ref):
    blk = in_ref[...]
    out_a_ref[...] = blk[:DIM_A, :].T
    out_b_ref[:, :DIM_B] = blk[DIM_A:, :].T


@jax.jit
def _tc_transpose(t_t):
    return pl.pallas_call(
        _transpose_body,
        grid=(pl.cdiv(VOCAB, TBLK),),
        in_specs=[pl.BlockSpec((DIM, TBLK), lambda i: (0, i))],
        out_specs=[
            pl.BlockSpec((TBLK, DIM_A), lambda i: (i, 0)),
            pl.BlockSpec((TBLK, DIM_A), lambda i: (i, 0)),
        ],
        out_shape=[
            jax.ShapeDtypeStruct((VOCAB, DIM_A), jnp.float32),
            jax.ShapeDtypeStruct((VOCAB, DIM_A), jnp.float32),
        ],
    )(t_t)


def _body(ta_hbm, tb_hbm, idxt_hbm, pa_hbm, pb_hbm, idx_v,
          buf_a0, buf_a1, buf_b0, buf_b1, gsem0, gsem1, wsem0, wsem1):
    bufs_a = (buf_a0, buf_a1)
    bufs_b = (buf_b0, buf_b1)
    gsems = (gsem0, gsem1)
    wsems = (wsem0, wsem1)

    c = lax.axis_index("c")
    s = lax.axis_index("s")
    wid = s * NC + c

    pltpu.sync_copy(idxt_hbm.at[:, pl.ds(wid * BPW, BPW)], idx_v)

    def start_gathers(b, li):
        pltpu.make_async_copy(
            ta_hbm.at[idx_v.at[li]], bufs_a[b], gsems[b]).start()
        pltpu.make_async_copy(
            tb_hbm.at[idx_v.at[li]], bufs_b[b], gsems[b]).start()

    def wait_gathers(b, li):
        pltpu.make_async_copy(
            ta_hbm.at[idx_v.at[li]], bufs_a[b], gsems[b]).wait()
        pltpu.make_async_copy(
            tb_hbm.at[idx_v.at[li]], bufs_b[b], gsems[b]).wait()

    for b in range(2):
        start_gathers(b, b)

    @pl.loop(0, SEQ // 2)
    def _outer(o):
        for b in range(2):
            li = o * 2 + b
            rows = pl.ds(li * BATCH + wid * BPW, BPW)
            wait_gathers(b, li)
            wa = pltpu.make_async_copy(bufs_a[b], pa_hbm.at[rows], wsems[b])
            wb = pltpu.make_async_copy(bufs_b[b], pb_hbm.at[rows], wsems[b])
            wa.start()
            wb.start()
            wa.wait()
            wb.wait()

            @pl.when(o < SEQ // 2 - 1)
            def _():
                start_gathers(b, li + 2)


@jax.jit
def _embed(idx_t, ta, tb):
    mesh = plsc.VectorSubcoreMesh(core_axis_name="c", subcore_axis_name="s")
    f = pl.kernel(
        _body,
        out_type=[
            jax.ShapeDtypeStruct((B_TOTAL, DIM_A), jnp.float32),
            jax.ShapeDtypeStruct((B_TOTAL, DIM_A), jnp.float32),
        ],
        mesh=mesh,
        scratch_types=[
            pltpu.VMEM((SEQ, BPW), jnp.int32),
            *[pltpu.VMEM((BPW, DIM_A), jnp.float32) for _ in range(4)],
            *[pltpu.SemaphoreType.DMA for _ in range(4)],
        ],
        compiler_params=pltpu.CompilerParams(use_tc_tiling_on_sc=False),
    )
    return f(ta, tb, idx_t)


def _out_transpose_body(a_ref, b_ref, o_ref):
    a2 = a_ref[0]
    b2 = b_ref[0]
    o_ref[0] = jnp.concatenate([a2, b2[:, :DIM_B]], axis=1).T


@jax.jit
def _tc_out(pa3, pb3):
    return pl.pallas_call(
        _out_transpose_body,
        grid=(SEQ, BATCH // OBB),
        in_specs=[
            pl.BlockSpec((1, OBB, DIM_A), lambda i, j: (i, j, 0)),
            pl.BlockSpec((1, OBB, DIM_A), lambda i, j: (i, j, 0)),
        ],
        out_specs=pl.BlockSpec((1, DIM, OBB), lambda i, j: (i, 0, j)),
        out_shape=jax.ShapeDtypeStruct((SEQ, DIM, BATCH), jnp.float32),
    )(pa3, pb3)


def kernel(indices, table):
    ta, tb = _tc_transpose(table.T)
    idx_t = indices.T.astype(jnp.int32)
    pa, pb = _embed(idx_t, ta, tb)
    out_t = _tc_out(pa.reshape(SEQ, BATCH, DIM_A), pb.reshape(SEQ, BATCH, DIM_A))
    return out_t.transpose(2, 0, 1)


# speedup vs baseline: 2.9901x; 1.2302x over previous
import jax
import jax.numpy as jnp
from jax import lax
from jax.experimental import pallas as pl
from jax.experimental.pallas import tpu as pltpu
from jax.experimental.pallas import tpu_sc as plsc

VOCAB = 100004
DIM = 200
BATCH = 4096
SEQ = 50

NC = 2
NS = 16
NW = NC * NS

BPW = BATCH // NW
B_TOTAL = BATCH * SEQ

TBLK = 2048
DIM_A = 128
DIM_B = DIM - DIM_A
OBB = 2048


def _transpose_body(in_ref, out_a_ref, out_b_ref):
    blk = in_ref[...]
    out_a_ref[...] = blk[:DIM_A, :].T
    out_b_ref[:, :DIM_B] = blk[DIM_A:, :].T


@jax.jit
def _tc_transpose(t_t):
    return pl.pallas_call(
        _transpose_body,
        grid=(pl.cdiv(VOCAB, TBLK),),
        in_specs=[pl.BlockSpec((DIM, TBLK), lambda i: (0, i))],
        out_specs=[
            pl.BlockSpec((TBLK, DIM_A), lambda i: (i, 0)),
            pl.BlockSpec((TBLK, DIM_A), lambda i: (i, 0)),
        ],
        out_shape=[
            jax.ShapeDtypeStruct((VOCAB, DIM_A), jnp.float32),
            jax.ShapeDtypeStruct((VOCAB, DIM_A), jnp.float32),
        ],
    )(t_t)


def _body(ta_hbm, tb_hbm, idxt_hbm, pa_hbm, pb_hbm, idx_v,
          buf_a0, buf_a1, buf_b0, buf_b1, gsem0, gsem1, wsem0, wsem1):
    bufs_a = (buf_a0, buf_a1)
    bufs_b = (buf_b0, buf_b1)
    gsems = (gsem0, gsem1)
    wsems = (wsem0, wsem1)

    c = lax.axis_index("c")
    s = lax.axis_index("s")
    wid = s * NC + c

    pltpu.sync_copy(idxt_hbm.at[:, pl.ds(wid * BPW, BPW)], idx_v)

    def start_gathers(b, li):
        pltpu.make_async_copy(
            ta_hbm.at[idx_v.at[li]], bufs_a[b], gsems[b]).start()
        pltpu.make_async_copy(
            tb_hbm.at[idx_v.at[li]], bufs_b[b], gsems[b]).start()

    def wait_gathers(b, li):
        pltpu.make_async_copy(
            ta_hbm.at[idx_v.at[li]], bufs_a[b], gsems[b]).wait()
        pltpu.make_async_copy(
            tb_hbm.at[idx_v.at[li]], bufs_b[b], gsems[b]).wait()

    for b in range(2):
        start_gathers(b, b)

    @pl.loop(0, SEQ // 2)
    def _outer(o):
        for b in range(2):
            li = o * 2 + b
            rows = pl.ds(li * BATCH + wid * BPW, BPW)
            wait_gathers(b, li)
            wa = pltpu.make_async_copy(bufs_a[b], pa_hbm.at[rows], wsems[b])
            wb = pltpu.make_async_copy(bufs_b[b], pb_hbm.at[rows], wsems[b])
            wa.start()
            wb.start()
            wa.wait()
            wb.wait()

            @pl.when(o < SEQ // 2 - 1)
            def _():
                start_gathers(b, li + 2)


@jax.jit
def _embed(idx_t, ta, tb):
    mesh = plsc.VectorSubcoreMesh(core_axis_name="c", subcore_axis_name="s")
    f = pl.kernel(
        _body,
        out_type=[
            jax.ShapeDtypeStruct((B_TOTAL, DIM_A), jnp.float32),
            jax.ShapeDtypeStruct((B_TOTAL, DIM_A), jnp.float32),
        ],
        mesh=mesh,
        scratch_types=[
            pltpu.VMEM((SEQ, BPW), jnp.int32),
            *[pltpu.VMEM((BPW, DIM_A), jnp.float32) for _ in range(4)],
            *[pltpu.SemaphoreType.DMA for _ in range(4)],
        ],
        compiler_params=pltpu.CompilerParams(use_tc_tiling_on_sc=False),
    )
    return f(ta, tb, idx_t)


def _out_transpose_body(a_ref, b_ref, o_ref):
    a2 = a_ref[0]
    b2 = b_ref[0]
    o_ref[0] = jnp.concatenate([a2, b2[:, :DIM_B]], axis=1).T


@jax.jit
def _tc_out(pa3, pb3):
    return pl.pallas_call(
        _out_transpose_body,
        grid=(SEQ, BATCH // OBB),
        in_specs=[
            pl.BlockSpec((1, OBB, DIM_A), lambda i, j: (i, j, 0)),
            pl.BlockSpec((1, OBB, DIM_A), lambda i, j: (i, j, 0)),
        ],
        out_specs=pl.BlockSpec((1, DIM, OBB), lambda i, j: (i, 0, j)),
        out_shape=jax.ShapeDtypeStruct((SEQ, DIM, BATCH), jnp.float32),
    )(pa3, pb3)


def kernel(indices, table):
    ta, tb = _tc_transpose(table.T)
    idx_t = indices.T.astype(jnp.int32)
    pa, pb = _embed(idx_t, ta, tb)
    out_t = _tc_out(pa.reshape(SEQ, BATCH, DIM_A), pb.reshape(SEQ, BATCH, DIM_A))
    return out_t.transpose(2, 0, 1)


# speedup vs baseline: 3.3314x; 1.1142x over previous
import jax
import jax.numpy as jnp
from jax import lax
from jax.experimental import pallas as pl
from jax.experimental.pallas import tpu as pltpu
from jax.experimental.pallas import tpu_sc as plsc

VOCAB = 100004
DIM = 200
BATCH = 4096
SEQ = 50

NC = 2
NS = 16
NW = NC * NS

BPW = BATCH // NW
B_TOTAL = BATCH * SEQ

TBLK = 4096
DIM_A = 128
DIM_B = DIM - DIM_A
OBB = 4096


def _transpose_body(in_ref, out_a_ref, out_b_ref):
    blk = in_ref[...]
    out_a_ref[...] = blk[:DIM_A, :].T
    out_b_ref[:, :DIM_B] = blk[DIM_A:, :].T


@jax.jit
def _tc_transpose(t_t):
    return pl.pallas_call(
        _transpose_body,
        grid=(pl.cdiv(VOCAB, TBLK),),
        in_specs=[pl.BlockSpec((DIM, TBLK), lambda i: (0, i))],
        out_specs=[
            pl.BlockSpec((TBLK, DIM_A), lambda i: (i, 0)),
            pl.BlockSpec((TBLK, DIM_A), lambda i: (i, 0)),
        ],
        out_shape=[
            jax.ShapeDtypeStruct((VOCAB, DIM_A), jnp.float32),
            jax.ShapeDtypeStruct((VOCAB, DIM_A), jnp.float32),
        ],
    )(t_t)


def _body(ta_hbm, tb_hbm, idxt_hbm, pa_hbm, pb_hbm, idx_v,
          buf_a0, buf_a1, buf_b0, buf_b1, gsem0, gsem1, wsem0, wsem1):
    bufs_a = (buf_a0, buf_a1)
    bufs_b = (buf_b0, buf_b1)
    gsems = (gsem0, gsem1)
    wsems = (wsem0, wsem1)

    c = lax.axis_index("c")
    s = lax.axis_index("s")
    wid = s * NC + c

    pltpu.sync_copy(idxt_hbm.at[:, pl.ds(wid * BPW, BPW)], idx_v)

    def start_gathers(b, li):
        pltpu.make_async_copy(
            ta_hbm.at[idx_v.at[li]], bufs_a[b], gsems[b]).start()
        pltpu.make_async_copy(
            tb_hbm.at[idx_v.at[li]], bufs_b[b], gsems[b]).start()

    def wait_gathers(b, li):
        pltpu.make_async_copy(
            ta_hbm.at[idx_v.at[li]], bufs_a[b], gsems[b]).wait()
        pltpu.make_async_copy(
            tb_hbm.at[idx_v.at[li]], bufs_b[b], gsems[b]).wait()

    for b in range(2):
        start_gathers(b, b)

    @pl.loop(0, SEQ // 2)
    def _outer(o):
        for b in range(2):
            li = o * 2 + b
            rows = pl.ds(li * BATCH + wid * BPW, BPW)
            wait_gathers(b, li)
            wa = pltpu.make_async_copy(bufs_a[b], pa_hbm.at[rows], wsems[b])
            wb = pltpu.make_async_copy(bufs_b[b], pb_hbm.at[rows], wsems[b])
            wa.start()
            wb.start()
            wa.wait()
            wb.wait()

            @pl.when(o < SEQ // 2 - 1)
            def _():
                start_gathers(b, li + 2)


@jax.jit
def _embed(idx_t, ta, tb):
    mesh = plsc.VectorSubcoreMesh(core_axis_name="c", subcore_axis_name="s")
    f = pl.kernel(
        _body,
        out_type=[
            jax.ShapeDtypeStruct((B_TOTAL, DIM_A), jnp.float32),
            jax.ShapeDtypeStruct((B_TOTAL, DIM_A), jnp.float32),
        ],
        mesh=mesh,
        scratch_types=[
            pltpu.VMEM((SEQ, BPW), jnp.int32),
            *[pltpu.VMEM((BPW, DIM_A), jnp.float32) for _ in range(4)],
            *[pltpu.SemaphoreType.DMA for _ in range(4)],
        ],
        compiler_params=pltpu.CompilerParams(use_tc_tiling_on_sc=False),
    )
    return f(ta, tb, idx_t)


def _out_transpose_body(a_ref, b_ref, o_ref):
    a2 = a_ref[0]
    b2 = b_ref[0]
    o_ref[0] = jnp.concatenate([a2, b2[:, :DIM_B]], axis=1).T


@jax.jit
def _tc_out(pa3, pb3):
    return pl.pallas_call(
        _out_transpose_body,
        grid=(SEQ, BATCH // OBB),
        in_specs=[
            pl.BlockSpec((1, OBB, DIM_A), lambda i, j: (i, j, 0)),
            pl.BlockSpec((1, OBB, DIM_A), lambda i, j: (i, j, 0)),
        ],
        out_specs=pl.BlockSpec((1, DIM, OBB), lambda i, j: (i, 0, j)),
        out_shape=jax.ShapeDtypeStruct((SEQ, DIM, BATCH), jnp.float32),
    )(pa3, pb3)


def kernel(indices, table):
    ta, tb = _tc_transpose(table.T)
    idx_t = indices.T.astype(jnp.int32)
    pa, pb = _embed(idx_t, ta, tb)
    out_t = _tc_out(pa.reshape(SEQ, BATCH, DIM_A), pb.reshape(SEQ, BATCH, DIM_A))
    return out_t.transpose(2, 0, 1)
